# baseline (speedup 1.0000x reference)
import functools

import jax
import jax.numpy as jnp
import numpy as np
from jax import lax
from jax.experimental import pallas as pl
from jax.experimental.pallas import tpu as pltpu
from jax.experimental.pallas import tpu_sc as plsc

_NUM_LEVELS = 16
_LEVEL_DIM = 2
_BASE_RES = 16
_HASHMAP_SIZE = 2 ** 19
_N_POINTS = 262144
_OUT_DIM = _NUM_LEVELS * _LEVEL_DIM
_TBL = _NUM_LEVELS * _HASHMAP_SIZE
_TBL_ELEMS = _TBL * _LEVEL_DIM

_L = 16
_NC = 2
_NS = 16
_NW = _NC * _NS
_PTS_PER_W = _N_POINTS // _NW
_CHUNK = 512
_NCHUNK = _PTS_PER_W // _CHUNK
_NSTEP = _CHUNK // _L
_IDX_MINOR = 8 * _L
_NIDX = _CHUNK * 8

_RF_PER_TILE = _TBL_ELEMS // _NS
_RF_CHUNK = 4096
_RF_ITERS = _RF_PER_TILE // _RF_CHUNK
_RF_BLOCKS = _RF_CHUNK // 256

_P1 = np.int32(-1640531535)
_P2 = np.int32(805459861)
_MASK = np.int32(_HASHMAP_SIZE - 1)
_CLIP_HI = np.float32(1.0 - 1e-6)


def _encoder_body(xt_hbm, emb_hbm, out_hbm, tmp_hbm, xv, ibufa, ibufb, wbufa,
                  wbufb, rowsa, rowsb, acc, sema, semb, semc, semd):
    cid = lax.axis_index("c")
    sid = lax.axis_index("s")
    wid = sid * _NC + cid
    base_pt = wid * _PTS_PER_W
    iota = lax.iota(jnp.int32, _L)
    zero = iota * np.int32(0)
    one = zero + np.int32(1)
    cid_tbl = cid * np.int32(_TBL)

    _rf_insem = {id(wbufa): sema, id(wbufb): semb}
    _rf_outsem = {id(rowsa): semc, id(rowsb): semd}

    def rf_in(k, wbuf):
        src_off = sid * _RF_PER_TILE + k * _RF_CHUNK
        return pltpu.async_copy(emb_hbm.at[pl.ds(src_off, _RF_CHUNK)], wbuf,
                                _rf_insem[id(wbuf)])

    def rf_out(k, rows):
        src_off = sid * _RF_PER_TILE + k * _RF_CHUNK
        return pltpu.async_copy(
            rows.at[pl.ds(0, _RF_CHUNK // 2), :],
            tmp_hbm.at[pl.ds(cid_tbl + src_off // 2, _RF_CHUNK // 2), :],
            _rf_outsem[id(rows)])

    def rf_wait_in(wbuf):
        pltpu.make_async_copy(emb_hbm.at[pl.ds(0, _RF_CHUNK)], wbuf,
                              _rf_insem[id(wbuf)]).wait()

    def rf_wait_out(rows):
        pltpu.make_async_copy(
            rows.at[pl.ds(0, _RF_CHUNK // 2), :],
            tmp_hbm.at[pl.ds(0, _RF_CHUNK // 2), :],
            _rf_outsem[id(rows)]).wait()

    def rf_permute(wbuf, rows):
        def blk_body(blk, carry1):
            sbase = blk * 256
            dbase = blk * 128
            for wg in range(8):
                v0 = wbuf[pl.ds(sbase + wg * _L, _L)]
                v1 = wbuf[pl.ds(sbase + 128 + wg * _L, _L)]
                rid = iota + (dbase + wg * _L)
                plsc.store_scatter(rows, [rid, zero], v0)
                plsc.store_scatter(rows, [rid, one], v1)
            return carry1

        lax.fori_loop(np.int32(0), np.int32(_RF_BLOCKS), blk_body,
                      np.int32(0), unroll=False)

    rf_in(np.int32(0), wbufa)

    def rf_body(k2, carry0):
        k = k2 * 2
        rf_in(k + 1, wbufb)

        @pl.when(k2 > np.int32(0))
        def _():
            rf_wait_out(rowsa)

        rf_wait_in(wbufa)
        rf_permute(wbufa, rowsa)
        rf_out(k, rowsa)

        @pl.when(k2 < np.int32(_RF_ITERS // 2 - 1))
        def _():
            rf_in(k + 2, wbufa)

        @pl.when(k2 > np.int32(0))
        def _():
            rf_wait_out(rowsb)

        rf_wait_in(wbufb)
        rf_permute(wbufb, rowsb)
        rf_out(k + 1, rowsb)
        return carry0

    lax.fori_loop(np.int32(0), np.int32(_RF_ITERS // 2), rf_body, np.int32(0),
                  unroll=False)
    rf_wait_out(rowsa)
    rf_wait_out(rowsb)
    plsc.subcore_barrier()

    def chunk_body(c, carry0):
        cbase = base_pt + c * _CHUNK
        pltpu.sync_copy(xt_hbm.at[:, pl.ds(cbase, _CHUNK)], xv)

        def step_a(l, ibuf, wbuf):
            res_f = np.float32(_BASE_RES << l)
            lvl_c = np.int32(l)
            lvl_or = cid_tbl + np.int32(l << 19)

            def body(s, carry2):
                sl = pl.ds(s * _L, _L)
                x0 = xv[0, sl]
                x1 = xv[1, sl]
                x2 = xv[2, sl]

                def coords(xc):
                    t = xc * np.float32(0.5) + np.float32(0.5)
                    t = jnp.minimum(jnp.maximum(t, np.float32(0.0)), _CLIP_HI)
                    sc = t * res_f
                    fi = sc.astype(jnp.int32)
                    fr = sc - fi.astype(jnp.float32)
                    return fi, fr

                f0, r0 = coords(x0)
                f1, r1 = coords(x1)
                f2, r2 = coords(x2)

                g1 = f1 * _P1
                g2 = f2 * _P2
                n0 = (f0, f0 + np.int32(1))
                h1 = (g1, g1 + _P1)
                h2 = (g2 ^ lvl_c, (g2 + _P2) ^ lvl_c)
                w0 = (np.float32(1.0) - r0, r0)
                w1 = (np.float32(1.0) - r1, r1)
                w2 = (np.float32(1.0) - r2, r2)

                for dy in range(2):
                    for dz in range(2):
                        wyz = w1[dy] * w2[dz]
                        hyz = h1[dy] ^ h2[dz]
                        for dx in range(2):
                            j = dx * 4 + dy * 2 + dz
                            idx = ((n0[dx] ^ hyz) & _MASK) | lvl_or
                            ibuf[pl.ds(s * _IDX_MINOR + j * _L, _L)] = idx
                            wbuf[pl.ds(s * _IDX_MINOR + j * _L, _L)] = (
                                w0[dx] * wyz)
                return carry2

            lax.fori_loop(np.int32(0), np.int32(_NSTEP), body, np.int32(0),
                          unroll=False)

        def step_b(l, wbuf, rows):
            rr = l >> 2
            obase = (l & 3) * 256

            def body(s, carry2):
                rbase = s * _IDX_MINOR
                a0 = None
                a1 = None
                for j in range(8):
                    rid = iota + (rbase + j * _L)
                    e0 = plsc.load_gather(rows, [rid, zero])
                    e1 = plsc.load_gather(rows, [rid, one])
                    w = wbuf[pl.ds(rbase + j * _L, _L)]
                    if j == 0:
                        a0 = e0 * w
                        a1 = e1 * w
                    else:
                        a0 = a0 + e0 * w
                        a1 = a1 + e1 * w
                cc = lax.shift_right_logical(s, np.int32(3))
                wb = obase + (s & np.int32(7)) * _L
                acc[rr, cc, pl.ds(wb, _L)] = a0
                acc[rr, cc, pl.ds(wb + np.int32(128), _L)] = a1
                return carry2

            lax.fori_loop(np.int32(0), np.int32(_NSTEP), body, np.int32(0),
                          unroll=False)

        bufs = ((ibufa, wbufa, rowsa, sema), (ibufb, wbufb, rowsb, sema))
        cps = [None] * _NUM_LEVELS
        for l in range(2):
            ibuf, wbuf, rows, sem = bufs[l % 2]
            step_a(l, ibuf, wbuf)
            cps[l] = pltpu.async_copy(tmp_hbm.at[ibuf], rows, sem)
        for l in range(_NUM_LEVELS):
            ibuf, wbuf, rows, sem = bufs[l % 2]
            cps[l].wait()
            step_b(l, wbuf, rows)
            if l + 2 < _NUM_LEVELS:
                step_a(l + 2, ibuf, wbuf)
                cps[l + 2] = pltpu.async_copy(tmp_hbm.at[ibuf], rows, sem)

        cb_abs = wid * (_PTS_PER_W // 128) + c * (_CHUNK // 128)
        pltpu.sync_copy(acc, out_hbm.at[:, pl.ds(cb_abs, _CHUNK // 128), :])
        return carry0

    lax.fori_loop(np.int32(0), np.int32(_NCHUNK), chunk_body, np.int32(0),
                  unroll=False)


_encoder = functools.partial(
    pl.kernel,
    out_type=(
        jax.ShapeDtypeStruct((4, _N_POINTS // 128, 1024), jnp.float32),
        jax.ShapeDtypeStruct((_NC * _TBL, _LEVEL_DIM), jnp.float32),
    ),
    mesh=plsc.VectorSubcoreMesh(core_axis_name="c", subcore_axis_name="s"),
    scratch_types=[
        pltpu.VMEM((3, _CHUNK), jnp.float32),
        pltpu.VMEM((_NIDX,), jnp.int32),
        pltpu.VMEM((_NIDX,), jnp.int32),
        pltpu.VMEM((_NIDX,), jnp.float32),
        pltpu.VMEM((_NIDX,), jnp.float32),
        pltpu.VMEM((_NIDX, _LEVEL_DIM), jnp.float32),
        pltpu.VMEM((_NIDX, _LEVEL_DIM), jnp.float32),
        pltpu.VMEM((4, _CHUNK // 128, 1024), jnp.float32),
        pltpu.SemaphoreType.DMA,
        pltpu.SemaphoreType.DMA,
        pltpu.SemaphoreType.DMA,
        pltpu.SemaphoreType.DMA,
    ],
    compiler_params=pltpu.CompilerParams(needs_layout_passes=False,
                                         use_tc_tiling_on_sc=False),
)(_encoder_body)


def kernel(x, emb):
    with jax.enable_x64(False):
        xt = x.astype(jnp.float32).T
        embf = (emb.astype(jnp.float32)
                .reshape(_NUM_LEVELS, _HASHMAP_SIZE // 128, 128, _LEVEL_DIM)
                .transpose(0, 1, 3, 2)
                .reshape(_TBL_ELEMS))
        out4, _ = _encoder(xt, embf)
        out4 = out4.reshape(4, _N_POINTS // 128, 8, 128)
        return out4.transpose(1, 3, 0, 2).reshape(_N_POINTS, _OUT_DIM)


# speedup vs baseline: 1.0183x; 1.0183x over previous
import functools

import jax
import jax.numpy as jnp
import numpy as np
from jax import lax
from jax.experimental import pallas as pl
from jax.experimental.pallas import tpu as pltpu
from jax.experimental.pallas import tpu_sc as plsc

_NUM_LEVELS = 16
_LEVEL_DIM = 2
_BASE_RES = 16
_HASHMAP_SIZE = 2 ** 19
_N_POINTS = 262144
_OUT_DIM = _NUM_LEVELS * _LEVEL_DIM
_TBL = _NUM_LEVELS * _HASHMAP_SIZE
_TBL_ELEMS = _TBL * _LEVEL_DIM

_L = 16
_NC = 2
_NS = 16
_NW = _NC * _NS
_PTS_PER_W = _N_POINTS // _NW
_CHUNK = 512
_NCHUNK = _PTS_PER_W // _CHUNK
_NSTEP = _CHUNK // _L
_IDX_MINOR = 8 * _L
_NIDX = _CHUNK * 8

_RF_PER_TILE = _TBL_ELEMS // _NS
_RF_CHUNK = 4096
_RF_ITERS = _RF_PER_TILE // _RF_CHUNK
_RF_BLOCKS = _RF_CHUNK // 256

_P1 = np.int32(-1640531535)
_P2 = np.int32(805459861)
_MASK = np.int32(_HASHMAP_SIZE - 1)
_CLIP_HI = np.float32(1.0 - 1e-6)


def _encoder_body(xt_hbm, emb_hbm, out_hbm, tmp_hbm, xv, ibufa, ibufb, wbufa,
                  wbufb, rowsa, rowsb, acc, sema, semb, semc, semd):
    cid = lax.axis_index("c")
    sid = lax.axis_index("s")
    wid = sid * _NC + cid
    base_pt = wid * _PTS_PER_W
    iota = lax.iota(jnp.int32, _L)
    zero = iota * np.int32(0)
    one = zero + np.int32(1)
    cid_tbl = cid * np.int32(_TBL)

    _rf_insem = {id(wbufa): sema, id(wbufb): semb}
    _rf_outsem = {id(rowsa): semc, id(rowsb): semd}

    def rf_in(k, wbuf):
        src_off = sid * _RF_PER_TILE + k * _RF_CHUNK
        return pltpu.async_copy(emb_hbm.at[pl.ds(src_off, _RF_CHUNK)], wbuf,
                                _rf_insem[id(wbuf)])

    def rf_out(k, rows):
        src_off = sid * _RF_PER_TILE + k * _RF_CHUNK
        return pltpu.async_copy(
            rows.at[pl.ds(0, _RF_CHUNK // 2), :],
            tmp_hbm.at[pl.ds(cid_tbl + src_off // 2, _RF_CHUNK // 2), :],
            _rf_outsem[id(rows)])

    def rf_wait_in(wbuf):
        pltpu.make_async_copy(emb_hbm.at[pl.ds(0, _RF_CHUNK)], wbuf,
                              _rf_insem[id(wbuf)]).wait()

    def rf_wait_out(rows):
        pltpu.make_async_copy(
            rows.at[pl.ds(0, _RF_CHUNK // 2), :],
            tmp_hbm.at[pl.ds(0, _RF_CHUNK // 2), :],
            _rf_outsem[id(rows)]).wait()

    def rf_permute(wbuf, rows):
        def blk_body(blk, carry1):
            sbase = blk * 256
            dbase = blk * 128
            for wg in range(8):
                v0 = wbuf[pl.ds(sbase + wg * _L, _L)]
                v1 = wbuf[pl.ds(sbase + 128 + wg * _L, _L)]
                rid = iota + (dbase + wg * _L)
                plsc.store_scatter(rows, [rid, zero], v0)
                plsc.store_scatter(rows, [rid, one], v1)
            return carry1

        lax.fori_loop(np.int32(0), np.int32(_RF_BLOCKS), blk_body,
                      np.int32(0), unroll=False)

    rf_in(np.int32(0), wbufa)

    def rf_body(k2, carry0):
        k = k2 * 2
        rf_in(k + 1, wbufb)

        @pl.when(k2 > np.int32(0))
        def _():
            rf_wait_out(rowsa)

        rf_wait_in(wbufa)
        rf_permute(wbufa, rowsa)
        rf_out(k, rowsa)

        @pl.when(k2 < np.int32(_RF_ITERS // 2 - 1))
        def _():
            rf_in(k + 2, wbufa)

        @pl.when(k2 > np.int32(0))
        def _():
            rf_wait_out(rowsb)

        rf_wait_in(wbufb)
        rf_permute(wbufb, rowsb)
        rf_out(k + 1, rowsb)
        return carry0

    lax.fori_loop(np.int32(0), np.int32(_RF_ITERS // 2), rf_body, np.int32(0),
                  unroll=False)
    rf_wait_out(rowsa)
    rf_wait_out(rowsb)
    plsc.subcore_barrier()

    def chunk_body(c, carry0):
        cbase = base_pt + c * _CHUNK
        pltpu.sync_copy(xt_hbm.at[:, pl.ds(cbase, _CHUNK)], xv)

        def step_a(l, ibuf, wbuf):
            res_f = np.float32(_BASE_RES << l)
            lvl_c = np.int32(l)
            lvl_or = cid_tbl + np.int32(l << 19)

            def body(s, carry2):
                sl = pl.ds(s * _L, _L)
                x0 = xv[0, sl]
                x1 = xv[1, sl]
                x2 = xv[2, sl]

                def coords(xc):
                    t = xc * np.float32(0.5) + np.float32(0.5)
                    t = jnp.minimum(jnp.maximum(t, np.float32(0.0)), _CLIP_HI)
                    sc = t * res_f
                    fi = sc.astype(jnp.int32)
                    fr = sc - fi.astype(jnp.float32)
                    return fi, fr

                f0, r0 = coords(x0)
                f1, r1 = coords(x1)
                f2, r2 = coords(x2)

                g1 = f1 * _P1
                g2 = f2 * _P2
                n0 = (f0, f0 + np.int32(1))
                h1 = (g1, g1 + _P1)
                h2 = (g2 ^ lvl_c, (g2 + _P2) ^ lvl_c)
                w0 = (np.float32(1.0) - r0, r0)
                w1 = (np.float32(1.0) - r1, r1)
                w2 = (np.float32(1.0) - r2, r2)

                for dy in range(2):
                    for dz in range(2):
                        wyz = w1[dy] * w2[dz]
                        hyz = h1[dy] ^ h2[dz]
                        for dx in range(2):
                            j = dx * 4 + dy * 2 + dz
                            idx = ((n0[dx] ^ hyz) & _MASK) | lvl_or
                            ibuf[pl.ds(s * _IDX_MINOR + j * _L, _L)] = idx
                            wbuf[pl.ds(s * _IDX_MINOR + j * _L, _L)] = (
                                w0[dx] * wyz)
                return carry2

            lax.fori_loop(np.int32(0), np.int32(_NSTEP), body, np.int32(0),
                          unroll=False)

        def step_b(l, wbuf, rows):
            rr = l >> 2
            obase = (l & 3) * 256

            def body(s, carry2):
                rbase = s * _IDX_MINOR
                a0 = None
                a1 = None
                for j in range(8):
                    rid = iota + (rbase + j * _L)
                    e0 = plsc.load_gather(rows, [rid, zero])
                    e1 = plsc.load_gather(rows, [rid, one])
                    w = wbuf[pl.ds(rbase + j * _L, _L)]
                    if j == 0:
                        a0 = e0 * w
                        a1 = e1 * w
                    else:
                        a0 = a0 + e0 * w
                        a1 = a1 + e1 * w
                cc = lax.shift_right_logical(s, np.int32(3))
                wb = obase + (s & np.int32(7)) * _L
                acc[rr, cc, pl.ds(wb, _L)] = a0
                acc[rr, cc, pl.ds(wb + np.int32(128), _L)] = a1
                return carry2

            lax.fori_loop(np.int32(0), np.int32(_NSTEP), body, np.int32(0),
                          unroll=False)

        bufs = ((ibufa, wbufa, rowsa), (ibufb, wbufb, rowsb))
        step_a(0, ibufa, wbufa)
        cp = pltpu.async_copy(tmp_hbm.at[ibufa], rowsa, sema)
        for l in range(_NUM_LEVELS):
            ibuf, wbuf, rows = bufs[l % 2]
            ibuf2, wbuf2, rows2 = bufs[(l + 1) % 2]
            if l + 1 < _NUM_LEVELS:
                step_a(l + 1, ibuf2, wbuf2)
            cp.wait()
            if l + 1 < _NUM_LEVELS:
                cp = pltpu.async_copy(tmp_hbm.at[ibuf2], rows2, sema)
            step_b(l, wbuf, rows)

        cb_abs = wid * (_PTS_PER_W // 128) + c * (_CHUNK // 128)
        pltpu.sync_copy(acc, out_hbm.at[:, pl.ds(cb_abs, _CHUNK // 128), :])
        return carry0

    lax.fori_loop(np.int32(0), np.int32(_NCHUNK), chunk_body, np.int32(0),
                  unroll=False)


_encoder = functools.partial(
    pl.kernel,
    out_type=(
        jax.ShapeDtypeStruct((4, _N_POINTS // 128, 1024), jnp.float32),
        jax.ShapeDtypeStruct((_NC * _TBL, _LEVEL_DIM), jnp.float32),
    ),
    mesh=plsc.VectorSubcoreMesh(core_axis_name="c", subcore_axis_name="s"),
    scratch_types=[
        pltpu.VMEM((3, _CHUNK), jnp.float32),
        pltpu.VMEM((_NIDX,), jnp.int32),
        pltpu.VMEM((_NIDX,), jnp.int32),
        pltpu.VMEM((_NIDX,), jnp.float32),
        pltpu.VMEM((_NIDX,), jnp.float32),
        pltpu.VMEM((_NIDX, _LEVEL_DIM), jnp.float32),
        pltpu.VMEM((_NIDX, _LEVEL_DIM), jnp.float32),
        pltpu.VMEM((4, _CHUNK // 128, 1024), jnp.float32),
        pltpu.SemaphoreType.DMA,
        pltpu.SemaphoreType.DMA,
        pltpu.SemaphoreType.DMA,
        pltpu.SemaphoreType.DMA,
    ],
    compiler_params=pltpu.CompilerParams(needs_layout_passes=False,
                                         use_tc_tiling_on_sc=False),
)(_encoder_body)


def kernel(x, emb):
    with jax.enable_x64(False):
        xt = x.astype(jnp.float32).T
        embf = (emb.astype(jnp.float32)
                .reshape(_NUM_LEVELS, _HASHMAP_SIZE // 128, 128, _LEVEL_DIM)
                .transpose(0, 1, 3, 2)
                .reshape(_TBL_ELEMS))
        out4, _ = _encoder(xt, embf)
        out4 = out4.reshape(4, _N_POINTS // 128, 8, 128)
        return out4.transpose(1, 3, 0, 2).reshape(_N_POINTS, _OUT_DIM)
